# Initial kernel scaffold; baseline (speedup 1.0000x reference)
import jax
import jax.numpy as jnp
from jax.experimental import pallas as pl


def kernel(x, state, W_lin, b_lin, Wq, Wk, Wv, Wo, ln_g, ln_b, Wg1, bg1, Wg2, bg2, Wu, bu):
    raise NotImplementedError("write your pallas kernel here")


# speedup vs baseline: 59.0045x; 59.0045x over previous
import jax
import jax.numpy as jnp
from jax.experimental import pallas as pl
from jax.experimental.pallas import tpu as pltpu

_TOPK = 32
_INT_MIN = -(2 ** 31)


def _agrnn_body(x_ref, st_ref, Wlin_ref, blin_ref,
                Wq_t_ref, Wq_s_ref, Wk_t_ref, Wk_s_ref, Wv_t_ref, Wv_s_ref,
                Wo_l_ref, Wo_r_ref, g_l_ref, g_r_ref, bln_l_ref, bln_r_ref,
                Wg1_t_ref, Wg1_s_ref, bg1_ref, Wg2_t_ref, Wg2_s_ref, bg2_ref,
                Wu_a_ref, Wu_b_ref, Wu_c_ref, bu_ref,
                h_ref, A_ref):
    f32 = jnp.float32
    N = x_ref.shape[1]

    xb = x_ref[0]
    st = st_ref[0]

    def mm(a, b):
        return jnp.dot(a, b, preferred_element_type=f32)

    xh = mm(xb, Wlin_ref[...]) + blin_ref[...]
    q = mm(xh, Wq_t_ref[...]) + mm(st, Wq_s_ref[...])
    k = mm(xh, Wk_t_ref[...]) + mm(st, Wk_s_ref[...])
    v = mm(xh, Wv_t_ref[...]) + mm(st, Wv_s_ref[...])

    s = jax.lax.dot_general(q, k, (((1,), (1,)), ((), ())),
                            preferred_element_type=f32) * f32(0.125)

    bi = jax.lax.bitcast_convert_type(s, jnp.int32)
    xi = jnp.where(bi >= 0, bi, bi ^ jnp.int32(0x7FFFFFFF))

    cnt0 = jnp.sum((xi >= 0).astype(jnp.int32), axis=1, keepdims=True)
    p = jnp.where(cnt0 >= _TOPK, jnp.int32(0), jnp.int32(_INT_MIN))

    def _step(i, p):
        bit = jnp.int32(30) - i
        cand = p | jnp.left_shift(jnp.int32(1), bit)
        cnt = jnp.sum((xi >= cand).astype(jnp.int32), axis=1, keepdims=True)
        return jnp.where(cnt >= _TOPK, cand, p)

    p = jax.lax.fori_loop(0, 31, _step, p)

    gt = xi > p
    eq = xi == p
    c_gt = jnp.sum(gt.astype(jnp.int32), axis=1, keepdims=True)

    eqf = eq.astype(f32)
    r_ = jax.lax.broadcasted_iota(jnp.int32, (N, N), 0)
    c_ = jax.lax.broadcasted_iota(jnp.int32, (N, N), 1)
    tri = (r_ <= c_).astype(f32)
    cum = mm(eqf, tri)
    quota = (jnp.int32(_TOPK) - c_gt).astype(f32)
    mask = gt | (eq & (cum <= quota))

    rowmax = jnp.max(s, axis=1, keepdims=True)
    e = jnp.exp(s - rowmax) * mask.astype(f32)
    den = jnp.sum(e, axis=1, keepdims=True)
    attn = e / den

    ctx = mm(attn, v)
    pre_l = xh + mm(ctx, Wo_l_ref[...])
    pre_r = st + mm(ctx, Wo_r_ref[...])
    twoH = f32(2 * xh.shape[1])
    mu = (jnp.sum(pre_l, axis=1, keepdims=True)
          + jnp.sum(pre_r, axis=1, keepdims=True)) / twoH
    dl = pre_l - mu
    dr = pre_r - mu
    var = (jnp.sum(dl * dl, axis=1, keepdims=True)
           + jnp.sum(dr * dr, axis=1, keepdims=True)) / twoH
    inv = f32(1.0) / jnp.sqrt(var + f32(1e-6))
    xl = dl * inv * g_l_ref[...] + bln_l_ref[...]
    xr = dr * inv * g_r_ref[...] + bln_r_ref[...]

    g1 = mm(xl, Wg1_t_ref[...]) + mm(xr, Wg1_s_ref[...])
    z = jax.nn.sigmoid((g1 + mm(attn, g1)) * f32(0.5) + bg1_ref[...])
    g2 = mm(xl, Wg2_t_ref[...]) + mm(xr, Wg2_s_ref[...])
    r = jax.nn.sigmoid((g2 + mm(attn, g2)) * f32(0.5) + bg2_ref[...])
    zs = z * st
    u = (mm(xl, Wu_a_ref[...]) + mm(xr, Wu_b_ref[...]) + mm(zs, Wu_c_ref[...]))
    hc = jnp.tanh((u + mm(attn, u)) * f32(0.5) + bu_ref[...])

    h_ref[0] = r * st + (f32(1.0) - r) * hc
    A_ref[0] = attn.T


def kernel(x, state, W_lin, b_lin, Wq, Wk, Wv, Wo, ln_g, ln_b,
           Wg1, bg1, Wg2, bg2, Wu, bu):
    B, N, DIN = x.shape
    H = state.shape[-1]

    ops = [
        x, state, W_lin, b_lin.reshape(1, H),
        Wq[:H], Wq[H:], Wk[:H], Wk[H:], Wv[:H], Wv[H:],
        Wo[:, :H], Wo[:, H:],
        ln_g[:H].reshape(1, H), ln_g[H:].reshape(1, H),
        ln_b[:H].reshape(1, H), ln_b[H:].reshape(1, H),
        Wg1[:H], Wg1[H:], bg1.reshape(1, H),
        Wg2[:H], Wg2[H:], bg2.reshape(1, H),
        Wu[:H], Wu[H:2 * H], Wu[2 * H:], bu.reshape(1, H),
    ]

    def wspec(a):
        zeros = (0,) * a.ndim
        return pl.BlockSpec(a.shape, lambda b, _z=zeros: _z)

    in_specs = [
        pl.BlockSpec((1, N, DIN), lambda b: (b, 0, 0)),
        pl.BlockSpec((1, N, H), lambda b: (b, 0, 0)),
    ] + [wspec(a) for a in ops[2:]]

    h, A = pl.pallas_call(
        _agrnn_body,
        grid=(B,),
        in_specs=in_specs,
        out_specs=[
            pl.BlockSpec((1, N, H), lambda b: (b, 0, 0)),
            pl.BlockSpec((1, N, N), lambda b: (b, 0, 0)),
        ],
        out_shape=[
            jax.ShapeDtypeStruct((B, N, H), jnp.float32),
            jax.ShapeDtypeStruct((B, N, N), jnp.float32),
        ],
        compiler_params=pltpu.CompilerParams(
            dimension_semantics=("parallel",),
        ),
    )(*ops)
    return h, A


# speedup vs baseline: 92.3516x; 1.5652x over previous
import jax
import jax.numpy as jnp
from jax.experimental import pallas as pl
from jax.experimental.pallas import tpu as pltpu

_TOPK = 32
_INT_MIN = -(2 ** 31)


def _agrnn_body(x_ref, st_ref, Wlin_ref, blin_ref,
                Wq_t_ref, Wq_s_ref, Wk_t_ref, Wk_s_ref, Wv_t_ref, Wv_s_ref,
                Wo_l_ref, Wo_r_ref, g_l_ref, g_r_ref, bln_l_ref, bln_r_ref,
                Wg1_t_ref, Wg1_s_ref, bg1_ref, Wg2_t_ref, Wg2_s_ref, bg2_ref,
                Wu_a_ref, Wu_b_ref, Wu_c_ref, bu_ref,
                h_ref, A_ref):
    f32 = jnp.float32
    bf16 = jnp.bfloat16
    N = x_ref.shape[1]

    xb = x_ref[0]
    st = st_ref[0]

    def mm(a, b):
        return jnp.dot(a, b, preferred_element_type=f32)

    def mm_t(a, b):
        return jax.lax.dot_general(a, b, (((0,), (0,)), ((), ())),
                                   preferred_element_type=f32)

    xh = mm(xb, Wlin_ref[...]) + blin_ref[...]
    q = mm(xh, Wq_t_ref[...]) + mm(st, Wq_s_ref[...])
    k = mm(xh, Wk_t_ref[...]) + mm(st, Wk_s_ref[...])
    v = mm(xh, Wv_t_ref[...]) + mm(st, Wv_s_ref[...])
    q8 = q * f32(0.125)

    sT = jax.lax.dot_general(k, q8, (((1,), (1,)), ((), ())),
                             preferred_element_type=f32)

    bi = jax.lax.bitcast_convert_type(sT, jnp.int32)
    xiT = jnp.where(bi >= 0, bi, bi ^ jnp.int32(0x7FFFFFFF))

    cnt0 = jnp.sum(jnp.where(xiT >= 0, f32(1.0), f32(0.0)),
                   axis=0, keepdims=True)
    p = jnp.where(cnt0 >= f32(_TOPK), jnp.int32(0), jnp.int32(_INT_MIN))

    def _step(i, p):
        bit = jnp.int32(30) - i
        cand = p | jnp.left_shift(jnp.int32(1), bit)
        cnt = jnp.sum(jnp.where(xiT >= cand, f32(1.0), f32(0.0)),
                      axis=0, keepdims=True)
        return jnp.where(cnt >= f32(_TOPK), cand, p)

    p = jax.lax.fori_loop(0, 31, _step, p)

    gtT = xiT > p
    eqT = xiT == p
    c_gt = jnp.sum(jnp.where(gtT, f32(1.0), f32(0.0)), axis=0, keepdims=True)

    r_ = jax.lax.broadcasted_iota(jnp.int32, (N, N), 0)
    c_ = jax.lax.broadcasted_iota(jnp.int32, (N, N), 1)
    ltri = (r_ >= c_).astype(bf16)
    cumT = mm_t(ltri, eqT.astype(bf16))
    quota = f32(_TOPK) - c_gt
    maskT = gtT | (eqT & (cumT <= quota))

    colmax = jnp.max(sT, axis=0, keepdims=True)
    e = jnp.exp(sT - colmax) * jnp.where(maskT, f32(1.0), f32(0.0))
    den = jnp.sum(e, axis=0, keepdims=True)
    attnT = e / den

    ctx = mm_t(attnT, v)
    pre_l = xh + mm(ctx, Wo_l_ref[...])
    pre_r = st + mm(ctx, Wo_r_ref[...])
    twoH = f32(2 * xh.shape[1])
    mu = (jnp.sum(pre_l, axis=1, keepdims=True)
          + jnp.sum(pre_r, axis=1, keepdims=True)) / twoH
    dl = pre_l - mu
    dr = pre_r - mu
    var = (jnp.sum(dl * dl, axis=1, keepdims=True)
           + jnp.sum(dr * dr, axis=1, keepdims=True)) / twoH
    inv = f32(1.0) / jnp.sqrt(var + f32(1e-6))
    xl = dl * inv * g_l_ref[...] + bln_l_ref[...]
    xr = dr * inv * g_r_ref[...] + bln_r_ref[...]

    g1 = mm(xl, Wg1_t_ref[...]) + mm(xr, Wg1_s_ref[...])
    z = jax.nn.sigmoid((g1 + mm_t(attnT, g1)) * f32(0.5) + bg1_ref[...])
    g2 = mm(xl, Wg2_t_ref[...]) + mm(xr, Wg2_s_ref[...])
    r = jax.nn.sigmoid((g2 + mm_t(attnT, g2)) * f32(0.5) + bg2_ref[...])
    zs = z * st
    u = (mm(xl, Wu_a_ref[...]) + mm(xr, Wu_b_ref[...]) + mm(zs, Wu_c_ref[...]))
    hc = jnp.tanh((u + mm_t(attnT, u)) * f32(0.5) + bu_ref[...])

    h_ref[0] = r * st + (f32(1.0) - r) * hc
    A_ref[0] = attnT


def kernel(x, state, W_lin, b_lin, Wq, Wk, Wv, Wo, ln_g, ln_b,
           Wg1, bg1, Wg2, bg2, Wu, bu):
    B, N, DIN = x.shape
    H = state.shape[-1]

    ops = [
        x, state, W_lin, b_lin.reshape(1, H),
        Wq[:H], Wq[H:], Wk[:H], Wk[H:], Wv[:H], Wv[H:],
        Wo[:, :H], Wo[:, H:],
        ln_g[:H].reshape(1, H), ln_g[H:].reshape(1, H),
        ln_b[:H].reshape(1, H), ln_b[H:].reshape(1, H),
        Wg1[:H], Wg1[H:], bg1.reshape(1, H),
        Wg2[:H], Wg2[H:], bg2.reshape(1, H),
        Wu[:H], Wu[H:2 * H], Wu[2 * H:], bu.reshape(1, H),
    ]

    def wspec(a):
        zeros = (0,) * a.ndim
        return pl.BlockSpec(a.shape, lambda b, _z=zeros: _z)

    in_specs = [
        pl.BlockSpec((1, N, DIN), lambda b: (b, 0, 0)),
        pl.BlockSpec((1, N, H), lambda b: (b, 0, 0)),
    ] + [wspec(a) for a in ops[2:]]

    h, A = pl.pallas_call(
        _agrnn_body,
        grid=(B,),
        in_specs=in_specs,
        out_specs=[
            pl.BlockSpec((1, N, H), lambda b: (b, 0, 0)),
            pl.BlockSpec((1, N, N), lambda b: (b, 0, 0)),
        ],
        out_shape=[
            jax.ShapeDtypeStruct((B, N, H), jnp.float32),
            jax.ShapeDtypeStruct((B, N, N), jnp.float32),
        ],
        compiler_params=pltpu.CompilerParams(
            dimension_semantics=("parallel",),
        ),
    )(*ops)
    return h, A


# speedup vs baseline: 99.3715x; 1.0760x over previous
import jax
import jax.numpy as jnp
from jax.experimental import pallas as pl
from jax.experimental.pallas import tpu as pltpu

_TOPK = 32
_INT_MIN = -(2 ** 31)
_PB = 2


def _agrnn_body(x_ref, st_ref, Wlin_ref, blin_ref,
                Wq_t_ref, Wq_s_ref,
                Wo_ref, g_l_ref, g_r_ref, bln_l_ref, bln_r_ref,
                Wg_t_ref, Wg_s_ref, bg1_ref, bg2_ref,
                Wu_a_ref, Wu_b_ref, Wu_c_ref, bu_ref,
                h_ref, A_ref):
    f32 = jnp.float32
    bf16 = jnp.bfloat16
    N = x_ref.shape[1]
    DIN = x_ref.shape[2]
    H = st_ref.shape[2]

    xb = x_ref[...].reshape(_PB * N, DIN)
    st = st_ref[...].reshape(_PB * N, H)

    def mm(a, b):
        return jnp.dot(a, b, preferred_element_type=f32)

    def mm_t(a, b):
        return jax.lax.dot_general(a, b, (((0,), (0,)), ((), ())),
                                   preferred_element_type=f32)

    def mm_nt(a, b):
        return jax.lax.dot_general(a, b, (((1,), (1,)), ((), ())),
                                   preferred_element_type=f32)

    xh = mm(xb, Wlin_ref[...]) + blin_ref[...]
    qkv = mm(xh, Wq_t_ref[...]) + mm(st, Wq_s_ref[...])
    q8 = qkv[:, :H] * f32(0.125)
    k = qkv[:, H:2 * H]
    v = qkv[:, 2 * H:]

    sT = jnp.concatenate(
        [mm_nt(k[b * N:(b + 1) * N], q8[b * N:(b + 1) * N])
         for b in range(_PB)], axis=1)

    bi = jax.lax.bitcast_convert_type(sT, jnp.int32)
    xiT = jnp.where(bi >= 0, bi, bi ^ jnp.int32(0x7FFFFFFF))

    def _count_ge(cand):
        ones = jnp.where(xiT >= cand, f32(1.0), f32(0.0))
        return jnp.sum(ones, axis=0, keepdims=True)

    p = jnp.where(_count_ge(jnp.int32(0)) >= f32(_TOPK),
                  jnp.int32(0), jnp.int32(_INT_MIN))

    def _step(i, p):
        bit = jnp.int32(30) - i
        cand = p | jnp.left_shift(jnp.int32(1), bit)
        return jnp.where(_count_ge(cand) >= f32(_TOPK), cand, p)

    p = jax.lax.fori_loop(0, 31, _step, p)

    gtT = xiT > p
    eqT = xiT == p
    c_gt = jnp.sum(jnp.where(gtT, f32(1.0), f32(0.0)), axis=0, keepdims=True)

    r_ = jax.lax.broadcasted_iota(jnp.int32, (N, N), 0)
    c_ = jax.lax.broadcasted_iota(jnp.int32, (N, N), 1)
    ltri = (r_ >= c_).astype(bf16)
    cumT = mm_t(ltri, eqT.astype(bf16))
    quota = f32(_TOPK) - c_gt
    maskT = gtT | (eqT & (cumT <= quota))

    colmax = jnp.max(sT, axis=0, keepdims=True)
    e = jnp.exp(sT - colmax) * jnp.where(maskT, f32(1.0), f32(0.0))
    den = jnp.sum(e, axis=0, keepdims=True)
    attnT = e / den

    def agg(y):
        return jnp.concatenate(
            [mm_t(attnT[:, b * N:(b + 1) * N], y[b * N:(b + 1) * N])
             for b in range(_PB)], axis=0)

    ctx = agg(v)
    cwo = mm(ctx, Wo_ref[...])
    pre_l = xh + cwo[:, :H]
    pre_r = st + cwo[:, H:]
    twoH = f32(2 * H)
    mu = (jnp.sum(pre_l, axis=1, keepdims=True)
          + jnp.sum(pre_r, axis=1, keepdims=True)) / twoH
    dl = pre_l - mu
    dr = pre_r - mu
    var = (jnp.sum(dl * dl, axis=1, keepdims=True)
           + jnp.sum(dr * dr, axis=1, keepdims=True)) / twoH
    inv = f32(1.0) / jnp.sqrt(var + f32(1e-6))
    xl = dl * inv * g_l_ref[...] + bln_l_ref[...]
    xr = dr * inv * g_r_ref[...] + bln_r_ref[...]

    g12 = mm(xl, Wg_t_ref[...]) + mm(xr, Wg_s_ref[...])
    a12 = agg(g12)
    z = jax.nn.sigmoid((g12[:, :H] + a12[:, :H]) * f32(0.5) + bg1_ref[...])
    r = jax.nn.sigmoid((g12[:, H:] + a12[:, H:]) * f32(0.5) + bg2_ref[...])
    zs = z * st
    u = (mm(xl, Wu_a_ref[...]) + mm(xr, Wu_b_ref[...]) + mm(zs, Wu_c_ref[...]))
    hc = jnp.tanh((u + agg(u)) * f32(0.5) + bu_ref[...])

    h_ref[...] = (r * st + (f32(1.0) - r) * hc).reshape(_PB, N, H)
    for b in range(_PB):
        A_ref[b] = attnT[:, b * N:(b + 1) * N]


def kernel(x, state, W_lin, b_lin, Wq, Wk, Wv, Wo, ln_g, ln_b,
           Wg1, bg1, Wg2, bg2, Wu, bu):
    B, N, DIN = x.shape
    H = state.shape[-1]

    Wqkv_t = jnp.concatenate([Wq[:H], Wk[:H], Wv[:H]], axis=1)
    Wqkv_s = jnp.concatenate([Wq[H:], Wk[H:], Wv[H:]], axis=1)
    Wg_t = jnp.concatenate([Wg1[:H], Wg2[:H]], axis=1)
    Wg_s = jnp.concatenate([Wg1[H:], Wg2[H:]], axis=1)
    ops = [
        x, state, W_lin, b_lin.reshape(1, H),
        Wqkv_t, Wqkv_s,
        Wo,
        ln_g[:H].reshape(1, H), ln_g[H:].reshape(1, H),
        ln_b[:H].reshape(1, H), ln_b[H:].reshape(1, H),
        Wg_t, Wg_s, bg1.reshape(1, H), bg2.reshape(1, H),
        Wu[:H], Wu[H:2 * H], Wu[2 * H:], bu.reshape(1, H),
    ]

    def wspec(a):
        zeros = (0,) * a.ndim
        return pl.BlockSpec(a.shape, lambda b, _z=zeros: _z)

    in_specs = [
        pl.BlockSpec((_PB, N, DIN), lambda b: (b, 0, 0)),
        pl.BlockSpec((_PB, N, H), lambda b: (b, 0, 0)),
    ] + [wspec(a) for a in ops[2:]]

    h, A = pl.pallas_call(
        _agrnn_body,
        grid=(B // _PB,),
        in_specs=in_specs,
        out_specs=[
            pl.BlockSpec((_PB, N, H), lambda b: (b, 0, 0)),
            pl.BlockSpec((_PB, N, N), lambda b: (b, 0, 0)),
        ],
        out_shape=[
            jax.ShapeDtypeStruct((B, N, H), jnp.float32),
            jax.ShapeDtypeStruct((B, N, N), jnp.float32),
        ],
        compiler_params=pltpu.CompilerParams(
            dimension_semantics=("parallel",),
        ),
    )(*ops)
    return h, A


# speedup vs baseline: 103.3471x; 1.0400x over previous
import jax
import jax.numpy as jnp
from jax.experimental import pallas as pl
from jax.experimental.pallas import tpu as pltpu

_TOPK = 32
_INT_MIN = -(2 ** 31)
_PB = 4


def _agrnn_body(x_ref, st_ref, Wlin_ref, blin_ref,
                Wq_t_ref, Wq_s_ref,
                Wo_ref, g_l_ref, g_r_ref, bln_l_ref, bln_r_ref,
                Wg_t_ref, Wg_s_ref, bg1_ref, bg2_ref,
                Wu_a_ref, Wu_b_ref, Wu_c_ref, bu_ref,
                h_ref, A_ref):
    f32 = jnp.float32
    bf16 = jnp.bfloat16
    N = x_ref.shape[1]
    DIN = x_ref.shape[2]
    H = st_ref.shape[2]

    xb = x_ref[...].reshape(_PB * N, DIN)
    st = st_ref[...].reshape(_PB * N, H)

    def mm(a, b):
        return jnp.dot(a, b, preferred_element_type=f32)

    def mm_t(a, b):
        return jax.lax.dot_general(a, b, (((0,), (0,)), ((), ())),
                                   preferred_element_type=f32)

    def mm_nt(a, b):
        return jax.lax.dot_general(a, b, (((1,), (1,)), ((), ())),
                                   preferred_element_type=f32)

    xh = mm(xb, Wlin_ref[...]) + blin_ref[...]
    qkv = mm(xh, Wq_t_ref[...]) + mm(st, Wq_s_ref[...])
    q8 = qkv[:, :H] * f32(0.125)
    k = qkv[:, H:2 * H]
    v = qkv[:, 2 * H:]

    sT = jnp.concatenate(
        [mm_nt(k[b * N:(b + 1) * N], q8[b * N:(b + 1) * N])
         for b in range(_PB)], axis=1)

    bi = jax.lax.bitcast_convert_type(sT, jnp.int32)
    xiT = jnp.where(bi >= 0, bi, bi ^ jnp.int32(0x7FFFFFFF))

    def _count_ge(cand):
        ones = jnp.where(xiT >= cand, f32(1.0), f32(0.0))
        return jnp.sum(ones, axis=0, keepdims=True)

    p = jnp.where(_count_ge(jnp.int32(0)) >= f32(_TOPK),
                  jnp.int32(0), jnp.int32(_INT_MIN))

    def _step(i, p):
        bit = jnp.int32(30) - i
        cand = p | jnp.left_shift(jnp.int32(1), bit)
        return jnp.where(_count_ge(cand) >= f32(_TOPK), cand, p)

    p = jax.lax.fori_loop(0, 31, _step, p)

    gtT = xiT > p
    eqT = xiT == p
    c_gt = jnp.sum(jnp.where(gtT, f32(1.0), f32(0.0)), axis=0, keepdims=True)

    r_ = jax.lax.broadcasted_iota(jnp.int32, (N, N), 0)
    c_ = jax.lax.broadcasted_iota(jnp.int32, (N, N), 1)
    ltri = (r_ >= c_).astype(bf16)
    cumT = mm_t(ltri, eqT.astype(bf16))
    quota = f32(_TOPK) - c_gt
    maskT = gtT | (eqT & (cumT <= quota))

    colmax = jnp.max(sT, axis=0, keepdims=True)
    e = jnp.exp(sT - colmax) * jnp.where(maskT, f32(1.0), f32(0.0))
    den = jnp.sum(e, axis=0, keepdims=True)
    attnT = e / den

    def agg(y):
        return jnp.concatenate(
            [mm_t(attnT[:, b * N:(b + 1) * N], y[b * N:(b + 1) * N])
             for b in range(_PB)], axis=0)

    ctx = agg(v)
    cwo = mm(ctx, Wo_ref[...])
    pre_l = xh + cwo[:, :H]
    pre_r = st + cwo[:, H:]
    twoH = f32(2 * H)
    mu = (jnp.sum(pre_l, axis=1, keepdims=True)
          + jnp.sum(pre_r, axis=1, keepdims=True)) / twoH
    dl = pre_l - mu
    dr = pre_r - mu
    var = (jnp.sum(dl * dl, axis=1, keepdims=True)
           + jnp.sum(dr * dr, axis=1, keepdims=True)) / twoH
    inv = f32(1.0) / jnp.sqrt(var + f32(1e-6))
    xl = dl * inv * g_l_ref[...] + bln_l_ref[...]
    xr = dr * inv * g_r_ref[...] + bln_r_ref[...]

    g12 = mm(xl, Wg_t_ref[...]) + mm(xr, Wg_s_ref[...])
    a12 = agg(g12)
    z = jax.nn.sigmoid((g12[:, :H] + a12[:, :H]) * f32(0.5) + bg1_ref[...])
    r = jax.nn.sigmoid((g12[:, H:] + a12[:, H:]) * f32(0.5) + bg2_ref[...])
    zs = z * st
    u = (mm(xl, Wu_a_ref[...]) + mm(xr, Wu_b_ref[...]) + mm(zs, Wu_c_ref[...]))
    hc = jnp.tanh((u + agg(u)) * f32(0.5) + bu_ref[...])

    h_ref[...] = (r * st + (f32(1.0) - r) * hc).reshape(_PB, N, H)
    for b in range(_PB):
        A_ref[b] = attnT[:, b * N:(b + 1) * N]


def kernel(x, state, W_lin, b_lin, Wq, Wk, Wv, Wo, ln_g, ln_b,
           Wg1, bg1, Wg2, bg2, Wu, bu):
    B, N, DIN = x.shape
    H = state.shape[-1]

    Wqkv_t = jnp.concatenate([Wq[:H], Wk[:H], Wv[:H]], axis=1)
    Wqkv_s = jnp.concatenate([Wq[H:], Wk[H:], Wv[H:]], axis=1)
    Wg_t = jnp.concatenate([Wg1[:H], Wg2[:H]], axis=1)
    Wg_s = jnp.concatenate([Wg1[H:], Wg2[H:]], axis=1)
    ops = [
        x, state, W_lin, b_lin.reshape(1, H),
        Wqkv_t, Wqkv_s,
        Wo,
        ln_g[:H].reshape(1, H), ln_g[H:].reshape(1, H),
        ln_b[:H].reshape(1, H), ln_b[H:].reshape(1, H),
        Wg_t, Wg_s, bg1.reshape(1, H), bg2.reshape(1, H),
        Wu[:H], Wu[H:2 * H], Wu[2 * H:], bu.reshape(1, H),
    ]

    def wspec(a):
        zeros = (0,) * a.ndim
        return pl.BlockSpec(a.shape, lambda b, _z=zeros: _z)

    in_specs = [
        pl.BlockSpec((_PB, N, DIN), lambda b: (b, 0, 0)),
        pl.BlockSpec((_PB, N, H), lambda b: (b, 0, 0)),
    ] + [wspec(a) for a in ops[2:]]

    h, A = pl.pallas_call(
        _agrnn_body,
        grid=(B // _PB,),
        in_specs=in_specs,
        out_specs=[
            pl.BlockSpec((_PB, N, H), lambda b: (b, 0, 0)),
            pl.BlockSpec((_PB, N, N), lambda b: (b, 0, 0)),
        ],
        out_shape=[
            jax.ShapeDtypeStruct((B, N, H), jnp.float32),
            jax.ShapeDtypeStruct((B, N, N), jnp.float32),
        ],
        compiler_params=pltpu.CompilerParams(
            dimension_semantics=("parallel",),
        ),
    )(*ops)
    return h, A
